# Initial kernel scaffold; baseline (speedup 1.0000x reference)
import jax
import jax.numpy as jnp
from jax.experimental import pallas as pl


def kernel(input_ids, token_type_ids, embedding_table, token_type_table):
    raise NotImplementedError("write your pallas kernel here")


# speedup vs baseline: 3.8275x; 3.8275x over previous
import math

import jax
import jax.numpy as jnp
from jax import lax
from jax.experimental import pallas as pl
from jax.experimental.pallas import tpu as pltpu
from jax.experimental.pallas import tpu_sc as plsc

VOCAB = 100000
D = 64
S = 200
B = 1024
N = B * S
NW = 32
ROWS_PER_W = N // NW
IDX_LANES = 128
CHUNK = 640
NSTREAM = CHUNK // IDX_LANES
NCHUNK = ROWS_PER_W // CHUNK
SCALE = math.sqrt(D)


def _positional_encoding():
    pos = jnp.arange(S, dtype=jnp.float32)[:, None]
    div = jnp.exp(jnp.arange(0, D, 2, dtype=jnp.float32) * (-math.log(10000.0) / D))
    ang = pos * div[None, :]
    pe = jnp.zeros((S, D), dtype=jnp.float32)
    pe = pe.at[:, 0::2].set(jnp.sin(ang))
    pe = pe.at[:, 1::2].set(jnp.cos(ang))
    return pe


def _emb_kernel(table, ids, tts, pe_in, ttab_in, out,
                comb, tbuf, idxv, ttv, rowsv, sem):
    nc = 2
    wid = lax.axis_index("s") * nc + lax.axis_index("c")

    pltpu.sync_copy(pe_in, comb.at[pl.ds(0, S)])
    pltpu.sync_copy(pe_in, comb.at[pl.ds(S, S)])
    pltpu.sync_copy(ttab_in, tbuf)
    for j in range(D // 16):
        sl = pl.ds(16 * j, 16)
        t0 = tbuf[0, sl]
        t1 = tbuf[1, sl]

        def _add_body(s, carry):
            a, b = carry
            comb[s, sl] += a
            comb[S + s, sl] += b
            return carry

        lax.fori_loop(0, S, _add_body, (t0, t1))

    def _chunk_body(ch, carry):
        row0 = wid * ROWS_PER_W + ch * CHUNK
        pltpu.sync_copy(ids.at[pl.ds(row0, CHUNK)], idxv)
        pltpu.sync_copy(tts.at[pl.ds(row0, CHUNK)], ttv)
        copies = [
            pltpu.async_copy(
                table.at[idxv.at[pl.ds(k * IDX_LANES, IDX_LANES)]],
                rowsv.at[pl.ds(k * IDX_LANES, IDX_LANES)],
                sem,
            )
            for k in range(NSTREAM)
        ]
        for c in copies:
            c.wait()

        def _grp_body(i, c2):
            ttvec = ttv[pl.ds(i * 16, 16)]
            base_g = row0 + i * 16
            for li in range(16):
                key = ttvec[li] * S + lax.rem(base_g + li, S)
                r = i * 16 + li
                for j in range(D // 16):
                    sl = pl.ds(16 * j, 16)
                    rowsv[r, sl] = rowsv[r, sl] * SCALE + comb[key, sl]
            return c2

        lax.fori_loop(0, CHUNK // 16, _grp_body, 0)

        pltpu.sync_copy(rowsv, out.at[pl.ds(row0, CHUNK)])
        return carry

    lax.fori_loop(0, NCHUNK, _chunk_body, 0)


def kernel(input_ids, token_type_ids, embedding_table, token_type_table):
    ids = input_ids.astype(jnp.int32).reshape(N)
    tts = token_type_ids.astype(jnp.int32).reshape(N)
    pe = _positional_encoding()

    mesh = plsc.VectorSubcoreMesh(core_axis_name="c", subcore_axis_name="s")
    run = pl.kernel(
        _emb_kernel,
        mesh=mesh,
        out_type=jax.ShapeDtypeStruct((N, D), jnp.float32),
        compiler_params=pltpu.CompilerParams(use_tc_tiling_on_sc=False),
        scratch_types=[
            pltpu.VMEM((2 * S, D), jnp.float32),
            pltpu.VMEM((2, D), jnp.float32),
            pltpu.VMEM((CHUNK,), jnp.int32),
            pltpu.VMEM((CHUNK,), jnp.int32),
            pltpu.VMEM((CHUNK, D), jnp.float32),
            pltpu.SemaphoreType.DMA,
        ],
    )
    out = run(embedding_table, ids, tts, pe, token_type_table)
    return out.reshape(B, S, D)


# speedup vs baseline: 4.0087x; 1.0473x over previous
import math

import jax
import jax.numpy as jnp
from jax import lax
from jax.experimental import pallas as pl
from jax.experimental.pallas import tpu as pltpu
from jax.experimental.pallas import tpu_sc as plsc

VOCAB = 100000
D = 64
S = 200
B = 1024
N = B * S
NW = 32
ROWS_PER_W = N // NW
IDX_LANES = 128
CHUNK = 640
NSTREAM = CHUNK // IDX_LANES
NCHUNK = ROWS_PER_W // CHUNK
SCALE = math.sqrt(D)


def _positional_encoding():
    pos = jnp.arange(S, dtype=jnp.float32)[:, None]
    div = jnp.exp(jnp.arange(0, D, 2, dtype=jnp.float32) * (-math.log(10000.0) / D))
    ang = pos * div[None, :]
    pe = jnp.zeros((S, D), dtype=jnp.float32)
    pe = pe.at[:, 0::2].set(jnp.sin(ang))
    pe = pe.at[:, 1::2].set(jnp.cos(ang))
    return pe


def _emb_kernel(table, ids, tts, pe_in, ttab_in, out,
                comb, tbuf, idxv0, idxv1, ttv0, ttv1, rowsv0, rowsv1,
                gsem0, gsem1, wsem0, wsem1):
    nc = 2
    wid = lax.axis_index("s") * nc + lax.axis_index("c")
    base = wid * ROWS_PER_W

    idxv = (idxv0, idxv1)
    ttv = (ttv0, ttv1)
    rowsv = (rowsv0, rowsv1)
    gsem = (gsem0, gsem1)
    wsem = (wsem0, wsem1)

    def stage(ch):
        b = ch % 2
        row0 = base + ch * CHUNK
        pltpu.sync_copy(ids.at[pl.ds(row0, CHUNK)], idxv[b])
        pltpu.sync_copy(tts.at[pl.ds(row0, CHUNK)], ttv[b])
        for k in range(NSTREAM):
            sl = pl.ds(k * IDX_LANES, IDX_LANES)
            pltpu.async_copy(table.at[idxv[b].at[sl]], rowsv[b].at[sl], gsem[b])

    def wait_gathers(ch):
        b = ch % 2
        for k in range(NSTREAM):
            sl = pl.ds(k * IDX_LANES, IDX_LANES)
            pltpu.make_async_copy(table.at[idxv[b].at[sl]],
                                  rowsv[b].at[sl], gsem[b]).wait()

    def writeback(ch):
        b = ch % 2
        row0 = base + ch * CHUNK
        pltpu.async_copy(rowsv[b], out.at[pl.ds(row0, CHUNK)], wsem[b])

    def wait_writeback(ch):
        b = ch % 2
        row0 = base + ch * CHUNK
        pltpu.make_async_copy(rowsv[b], out.at[pl.ds(row0, CHUNK)],
                              wsem[b]).wait()

    def compute(ch):
        b = ch % 2
        row0 = base + ch * CHUNK
        rv, tv = rowsv[b], ttv[b]

        def _grp_body(i, c2):
            ttvec = tv[pl.ds(i * 16, 16)]
            base_g = row0 + i * 16
            for li in range(16):
                key = ttvec[li] * S + lax.rem(base_g + li, S)
                r = i * 16 + li
                for j in range(D // 16):
                    sl = pl.ds(16 * j, 16)
                    rv[r, sl] = rv[r, sl] * SCALE + comb[key, sl]
            return c2

        lax.fori_loop(0, CHUNK // 16, _grp_body, 0)

    stage(0)

    pltpu.sync_copy(pe_in, comb.at[pl.ds(0, S)])
    pltpu.sync_copy(pe_in, comb.at[pl.ds(S, S)])
    pltpu.sync_copy(ttab_in, tbuf)
    for j in range(D // 16):
        sl = pl.ds(16 * j, 16)
        t0 = tbuf[0, sl]
        t1 = tbuf[1, sl]

        def _add_body(s, carry):
            a, b2 = carry
            comb[s, sl] += a
            comb[S + s, sl] += b2
            return carry

        lax.fori_loop(0, S, _add_body, (t0, t1))

    for ch in range(NCHUNK):
        wait_gathers(ch)
        if ch + 1 < NCHUNK:
            if ch >= 1:
                wait_writeback(ch - 1)
            stage(ch + 1)
        compute(ch)
        writeback(ch)

    wait_writeback(NCHUNK - 2)
    wait_writeback(NCHUNK - 1)


def kernel(input_ids, token_type_ids, embedding_table, token_type_table):
    ids = input_ids.astype(jnp.int32).reshape(N)
    tts = token_type_ids.astype(jnp.int32).reshape(N)
    pe = _positional_encoding()

    mesh = plsc.VectorSubcoreMesh(core_axis_name="c", subcore_axis_name="s")
    run = pl.kernel(
        _emb_kernel,
        mesh=mesh,
        out_type=jax.ShapeDtypeStruct((N, D), jnp.float32),
        compiler_params=pltpu.CompilerParams(use_tc_tiling_on_sc=False),
        scratch_types=[
            pltpu.VMEM((2 * S, D), jnp.float32),
            pltpu.VMEM((2, D), jnp.float32),
            pltpu.VMEM((CHUNK,), jnp.int32),
            pltpu.VMEM((CHUNK,), jnp.int32),
            pltpu.VMEM((CHUNK,), jnp.int32),
            pltpu.VMEM((CHUNK,), jnp.int32),
            pltpu.VMEM((CHUNK, D), jnp.float32),
            pltpu.VMEM((CHUNK, D), jnp.float32),
            pltpu.SemaphoreType.DMA,
            pltpu.SemaphoreType.DMA,
            pltpu.SemaphoreType.DMA,
            pltpu.SemaphoreType.DMA,
        ],
    )
    out = run(embedding_table, ids, tts, pe, token_type_table)
    return out.reshape(B, S, D)


# speedup vs baseline: 4.9663x; 1.2389x over previous
import math

import jax
import jax.numpy as jnp
from jax import lax
from jax.experimental import pallas as pl
from jax.experimental.pallas import tpu as pltpu
from jax.experimental.pallas import tpu_sc as plsc

VOCAB = 100000
D = 64
S = 200
B = 1024
NW = 32
SBLK = 8
BBLK = 128
NUNITS = (S // SBLK) * (B // BBLK)
UMAX = -(-NUNITS // NW)
CPAD = D + 1
TPAD = 2 * D + 2
SCALE = math.sqrt(D)


def _positional_encoding():
    pos = jnp.arange(S, dtype=jnp.float32)[:, None]
    div = jnp.exp(jnp.arange(0, D, 2, dtype=jnp.float32) * (-math.log(10000.0) / D))
    ang = pos * div[None, :]
    pe = jnp.zeros((S, D), dtype=jnp.float32)
    pe = pe.at[:, 0::2].set(jnp.sin(ang))
    pe = pe.at[:, 1::2].set(jnp.cos(ang))
    return pe


def _emb_kernel(table, ids_t, tts_t, pe_in, ttab_in, out,
                comb, pebuf, tbuf, idxv, ttv, rows0, rows1, tile0, tile1,
                gsem0, gsem1, wsem0, wsem1):
    nc = 2
    wid = lax.axis_index("s") * nc + lax.axis_index("c")
    rows = (rows0, rows1)
    tiles = (tile0, tile1)
    gsem = (gsem0, gsem1)
    wsem = (wsem0, wsem1)
    iota16 = lax.iota(jnp.int32, 16)

    pltpu.sync_copy(pe_in, pebuf)
    pltpu.sync_copy(ttab_in, tbuf)
    for jq in range(D // 16):
        sl = pl.ds(16 * jq, 16)
        jvec0 = iota16 + 16 * jq
        t0 = tbuf[0, sl]
        t1 = tbuf[1, sl]

        def _add_body(s, carry):
            a, b2 = carry
            pv = pebuf[s, sl]
            plsc.store_scatter(comb, [jnp.full((16,), s, jnp.int32), jvec0],
                               pv + a)
            plsc.store_scatter(comb, [jnp.full((16,), S + s, jnp.int32), jvec0],
                               pv + b2)
            return carry

        lax.fori_loop(0, S, _add_body, (t0, t1))

    def _unit_body(u, carry):
        unit = wid + NW * u

        @pl.when(unit < NUNITS)
        def _run():
            s_base = pl.multiple_of((unit // (B // BBLK)) * SBLK, SBLK)
            b0 = pl.multiple_of((unit % (B // BBLK)) * BBLK, BBLK)
            bt = unit % (B // BBLK)

            pltpu.sync_copy(ids_t.at[pl.ds(s_base, SBLK), pl.ds(b0, BBLK)],
                            idxv)
            pltpu.sync_copy(tts_t.at[pl.ds(s_base, SBLK), pl.ds(b0, BBLK)],
                            ttv)

            def fire(si):
                pltpu.async_copy(table.at[idxv.at[si]], rows[si % 2],
                                 gsem[si % 2])

            def wait_g(si):
                pltpu.make_async_copy(table.at[idxv.at[si]], rows[si % 2],
                                      gsem[si % 2]).wait()

            def compute(si):
                s = s_base + si
                rv, tl = rows[si % 2], tiles[si % 2]

                def _bg_body(bg, c2):
                    bsl = pl.ds(bg * 16, 16)
                    ttvec = ttv[si, bsl]
                    kvec = ttvec * S + s
                    bvec = iota16 + bg * 16

                    def _jd_body(jq, c3):
                        for jr in range(4):
                            jvec = (jq * 4 + jr + iota16) & (D - 1)
                            evec = plsc.load_gather(rv, [bvec, jvec])
                            cvec = plsc.load_gather(comb, [kvec, jvec])
                            plsc.store_scatter(tl, [jvec, bvec],
                                               evec * SCALE + cvec)
                        return c3

                    lax.fori_loop(0, D // 4, _jd_body, 0)
                    return c2

                lax.fori_loop(0, BBLK // 16, _bg_body, 0)

            def writeback(si):
                s = s_base + si
                for jt in range(8):
                    pltpu.async_copy(
                        tiles[si % 2].at[pl.ds(jt * 8, 8), pl.ds(0, BBLK)],
                        out.at[s, jt, bt],
                        wsem[si % 2])

            def wait_wb(si):
                s = s_base + si
                for jt in range(8):
                    pltpu.make_async_copy(
                        tiles[si % 2].at[pl.ds(jt * 8, 8), pl.ds(0, BBLK)],
                        out.at[s, jt, bt],
                        wsem[si % 2]).wait()

            @pl.when(u > 0)
            def _drain():
                for p in range(2):
                    for jt in range(8):
                        pltpu.make_async_copy(
                            tiles[p].at[pl.ds(jt * 8, 8), pl.ds(0, BBLK)],
                            out.at[0, jt, 0],
                            wsem[p]).wait()

            fire(0)
            for si in range(SBLK):
                if si + 1 < SBLK:
                    fire(si + 1)
                wait_g(si)
                if si >= 2:
                    wait_wb(si - 2)
                compute(si)
                writeback(si)

        return carry

    lax.fori_loop(0, UMAX, _unit_body, 0)

    for p in range(2):
        for jt in range(8):
            pltpu.make_async_copy(
                tiles[p].at[pl.ds(jt * 8, 8), pl.ds(0, BBLK)],
                out.at[0, jt, 0],
                wsem[p]).wait()


def kernel(input_ids, token_type_ids, embedding_table, token_type_table):
    ids_t = input_ids.astype(jnp.int32).T
    tts_t = token_type_ids.astype(jnp.int32).T
    pe = _positional_encoding()

    mesh = plsc.VectorSubcoreMesh(core_axis_name="c", subcore_axis_name="s")
    run = pl.kernel(
        _emb_kernel,
        mesh=mesh,
        out_type=jax.ShapeDtypeStruct((S, 8, B // BBLK, 8, BBLK), jnp.float32),
        compiler_params=pltpu.CompilerParams(use_tc_tiling_on_sc=False,
                                             needs_layout_passes=False),
        scratch_types=[
            pltpu.VMEM((2 * S, CPAD), jnp.float32),
            pltpu.VMEM((S, D), jnp.float32),
            pltpu.VMEM((2, D), jnp.float32),
            pltpu.VMEM((SBLK, BBLK), jnp.int32),
            pltpu.VMEM((SBLK, BBLK), jnp.int32),
            pltpu.VMEM((BBLK, D), jnp.float32),
            pltpu.VMEM((BBLK, D), jnp.float32),
            pltpu.VMEM((D, TPAD), jnp.float32),
            pltpu.VMEM((D, TPAD), jnp.float32),
            pltpu.SemaphoreType.DMA,
            pltpu.SemaphoreType.DMA,
            pltpu.SemaphoreType.DMA,
            pltpu.SemaphoreType.DMA,
        ],
    )
    out = run(embedding_table, ids_t, tts_t, pe, token_type_table)
    x = jnp.transpose(out, (2, 4, 0, 1, 3))
    return x.reshape(B, S, D)
